# baseline (speedup 1.0000x reference)
import functools

import jax
import jax.numpy as jnp
from jax import lax
from jax.experimental import pallas as pl
from jax.experimental.pallas import tpu as pltpu
from jax.experimental.pallas import tpu_sc as plsc

LAT = 128
N_NODES_K = 10000
N_EDGES_K = 320000

NC, NS = 2, 16
NW = NC * NS
CHUNK = 128
N_EDGES_H = N_EDGES_K // 2
N_CHUNKS_H = N_EDGES_H // CHUNK
CPW_H = (N_CHUNKS_H + NW - 1) // NW
NFULL = N_NODES_K // CHUNK
NTAIL = N_NODES_K - NFULL * CHUNK
NSLICE = NFULL + 1
SPS = (NSLICE + NS - 1) // NS
ZROWS = 16
NZ = N_NODES_K // ZROWS
ZPS = (NZ + NS - 1) // NS

_SC_MESH = plsc.VectorSubcoreMesh(
    core_axis_name="c", subcore_axis_name="s", num_cores=NC, num_subcores=NS)


def _pre_body(x_ref, wa_ref, wb_ref, b1_ref, xa_ref, xb_ref):
    x = x_ref[...]
    xa_ref[...] = jnp.dot(x, wa_ref[...], preferred_element_type=jnp.float32)
    xb_ref[...] = (jnp.dot(x, wb_ref[...], preferred_element_type=jnp.float32)
                   + b1_ref[...])


_pre_call = pl.pallas_call(
    _pre_body,
    out_shape=(jax.ShapeDtypeStruct((N_NODES_K, LAT), jnp.float32),
               jax.ShapeDtypeStruct((N_NODES_K, LAT), jnp.float32)),
)


LROWS = 640
LTAIL = N_NODES_K - 15 * LROWS


PCH = CPW_H


def _gather_body(xa_hbm, xb_hbm, row_hbm, col_hbm, g_hbm,
                 xa_sh, ridx, cidx, b0, b1, s0, s1, sw):
    sid = lax.axis_index("s")
    wid = sid * NC + lax.axis_index("c")

    c0 = jnp.minimum(wid * CPW_H, N_CHUNKS_H - PCH)

    @pl.when(sid < NS - 1)
    def _():
        pltpu.sync_copy(xa_hbm.at[pl.ds(sid * LROWS, LROWS)],
                        xa_sh.at[pl.ds(sid * LROWS, LROWS)])

    @pl.when(sid == NS - 1)
    def _():
        pltpu.sync_copy(xa_hbm.at[pl.ds((NS - 1) * LROWS, LTAIL)],
                        xa_sh.at[pl.ds((NS - 1) * LROWS, LTAIL)])

    pltpu.sync_copy(row_hbm.at[pl.ds(c0 * CHUNK, PCH * CHUNK)], ridx)
    pltpu.sync_copy(col_hbm.at[pl.ds(c0 * CHUNK, PCH * CHUNK)], cidx)
    plsc.subcore_barrier()

    bufs = (b0, b1)
    sems = (s0, s1)

    def j_body(j, carry):
        adds = []
        for b in range(2):
            i = j * 2 + b
            ri = ridx.at[pl.ds(i * CHUNK, CHUNK)]
            ci = cidx.at[pl.ds(i * CHUNK, CHUNK)]
            pltpu.sync_copy(xa_sh.at[ri], bufs[b])
            adds.append(pltpu.async_copy(xb_hbm.at[ci], bufs[b],
                                         sems[b], add=True))
        adds[0].wait()
        base0 = (c0 + j * 2) * CHUNK
        w0 = pltpu.async_copy(b0, g_hbm.at[pl.ds(base0, CHUNK)], sw)
        adds[1].wait()
        pltpu.sync_copy(b1, g_hbm.at[pl.ds(base0 + CHUNK, CHUNK)])
        w0.wait()
        return carry

    lax.fori_loop(0, PCH // 2, j_body, 0)


_gather_call = pl.kernel(
    _gather_body,
    out_type=jax.ShapeDtypeStruct((N_EDGES_H, LAT), jnp.float32),
    mesh=_SC_MESH,
    scratch_types=(
        pltpu.VMEM_SHARED((N_NODES_K, LAT), jnp.float32),
        pltpu.VMEM((PCH * CHUNK,), jnp.int32),
        pltpu.VMEM((PCH * CHUNK,), jnp.int32),
        pltpu.VMEM((CHUNK, LAT), jnp.float32),
        pltpu.VMEM((CHUNK, LAT), jnp.float32),
        pltpu.SemaphoreType.DMA,
        pltpu.SemaphoreType.DMA,
        pltpu.SemaphoreType.DMA,
    ),
)


def _gather_dep_body(xa_hbm, xb_hbm, row_hbm, col_hbm, dep_hbm, g_hbm,
                     *scratch):
    _gather_body(xa_hbm, xb_hbm, row_hbm, col_hbm, g_hbm, *scratch)


_gather_dep_call = pl.kernel(
    _gather_dep_body,
    out_type=jax.ShapeDtypeStruct((N_EDGES_H, LAT), jnp.float32),
    mesh=_SC_MESH,
    scratch_types=(
        pltpu.VMEM_SHARED((N_NODES_K, LAT), jnp.float32),
        pltpu.VMEM((PCH * CHUNK,), jnp.int32),
        pltpu.VMEM((PCH * CHUNK,), jnp.int32),
        pltpu.VMEM((CHUNK, LAT), jnp.float32),
        pltpu.VMEM((CHUNK, LAT), jnp.float32),
        pltpu.SemaphoreType.DMA,
        pltpu.SemaphoreType.DMA,
        pltpu.SemaphoreType.DMA,
    ),
)


EBLK = 4000


def _edge_body(g_ref, e_ref, w1c_ref, w2_ref, b2_ref, out_ref):
    e = e_ref[...]
    h = jnp.dot(e.astype(jnp.bfloat16), w1c_ref[...].astype(jnp.bfloat16),
                preferred_element_type=jnp.float32)
    h = jnp.maximum(h + g_ref[...].astype(jnp.float32), 0.0)
    out_ref[...] = (e + jnp.dot(h.astype(jnp.bfloat16),
                                w2_ref[...].astype(jnp.bfloat16),
                                preferred_element_type=jnp.float32)
                    + b2_ref[...])


_edge_call = pl.pallas_call(
    _edge_body,
    grid=(N_EDGES_H // EBLK,),
    in_specs=[
        pl.BlockSpec((EBLK, LAT), lambda i: (i, 0)),
        pl.BlockSpec((EBLK, LAT), lambda i: (i, 0)),
        pl.BlockSpec((LAT, LAT), lambda i: (0, 0)),
        pl.BlockSpec((LAT, LAT), lambda i: (0, 0)),
        pl.BlockSpec((1, LAT), lambda i: (0, 0)),
    ],
    out_specs=pl.BlockSpec((EBLK, LAT), lambda i: (i, 0)),
    out_shape=jax.ShapeDtypeStruct((N_EDGES_H, LAT), jnp.float32),
    compiler_params=pltpu.CompilerParams(
        dimension_semantics=("parallel",)),
)


def _scatter_body(en_hbm, col2_hbm, out_hbm, acc_shared, cidx,
                  d0, d1, zbuf, t0, t1):
    cid = lax.axis_index("c")
    sid = lax.axis_index("s")
    wid = sid * NC + cid

    def zrow(r, carry):
        for j in range(LAT // 16):
            zbuf[r, pl.ds(j * 16, 16)] = jnp.zeros((16,), jnp.float32)
        return carry

    lax.fori_loop(0, ZROWS, zrow, 0)

    def zslice(i, carry):
        j = i * NS + sid

        @pl.when(j < NZ)
        def _():
            pltpu.sync_copy(zbuf, acc_shared.at[pl.ds(j * ZROWS, ZROWS)])

        return carry

    lax.fori_loop(0, ZPS, zslice, 0)

    c0 = jnp.minimum(wid * CPW_H, N_CHUNKS_H - CPW_H)
    off = wid * CPW_H - c0
    pltpu.sync_copy(col2_hbm.at[pl.ds(c0 * CHUNK, CPW_H * CHUNK)], cidx)
    plsc.subcore_barrier()

    dbufs = (d0, d1)
    dsems = (t0, t1)

    def j_body(j, carry):
        loads = []
        for b in range(2):
            c = wid * CPW_H + j * 2 + b
            base = jnp.minimum(c, N_CHUNKS_H - 1) * CHUNK
            loads.append(pltpu.async_copy(en_hbm.at[pl.ds(base, CHUNK)],
                                          dbufs[b], dsems[b]))
        for b in range(2):
            i = j * 2 + b
            c = wid * CPW_H + i
            loads[b].wait()

            @pl.when(c < N_CHUNKS_H)
            def _(b=b, i=i):
                pltpu.sync_copy(
                    dbufs[b],
                    acc_shared.at[cidx.at[pl.ds((off + i) * CHUNK, CHUNK)]],
                    add=True)

        return carry

    lax.fori_loop(0, (CPW_H + 1) // 2, j_body, 0)
    plsc.subcore_barrier()

    def oslice(i, carry):
        j = i * NS + sid

        @pl.when(j < NFULL)
        def _():
            pltpu.sync_copy(acc_shared.at[pl.ds(j * CHUNK, CHUNK)],
                            out_hbm.at[cid, pl.ds(j * CHUNK, CHUNK)])

        @pl.when(j == NFULL)
        def _():
            pltpu.sync_copy(acc_shared.at[pl.ds(NFULL * CHUNK, NTAIL)],
                            out_hbm.at[cid, pl.ds(NFULL * CHUNK, NTAIL)])

        return carry

    lax.fori_loop(0, SPS, oslice, 0)


_scatter_call = pl.kernel(
    _scatter_body,
    out_type=jax.ShapeDtypeStruct((NC, N_NODES_K, LAT), jnp.float32),
    mesh=_SC_MESH,
    scratch_types=(
        pltpu.VMEM_SHARED((N_NODES_K, LAT), jnp.float32),
        pltpu.VMEM((CPW_H * CHUNK,), jnp.int32),
        pltpu.VMEM((CHUNK, LAT), jnp.float32),
        pltpu.VMEM((CHUNK, LAT), jnp.float32),
        pltpu.VMEM((ZROWS, LAT), jnp.float32),
        pltpu.SemaphoreType.DMA,
        pltpu.SemaphoreType.DMA,
    ),
)


def _scatter_dep_body(en_hbm, col2_hbm, dep_hbm, out_hbm, *scratch):
    _scatter_body(en_hbm, col2_hbm, out_hbm, *scratch)


_scatter_dep_call = pl.kernel(
    _scatter_dep_body,
    out_type=jax.ShapeDtypeStruct((NC, N_NODES_K, LAT), jnp.float32),
    mesh=_SC_MESH,
    scratch_types=(
        pltpu.VMEM_SHARED((N_NODES_K, LAT), jnp.float32),
        pltpu.VMEM((CPW_H * CHUNK,), jnp.int32),
        pltpu.VMEM((CHUNK, LAT), jnp.float32),
        pltpu.VMEM((CHUNK, LAT), jnp.float32),
        pltpu.VMEM((ZROWS, LAT), jnp.float32),
        pltpu.SemaphoreType.DMA,
        pltpu.SemaphoreType.DMA,
    ),
)


def _node_body(x_ref, agga_ref, aggb_ref, w1a_ref, w1b_ref, b1_ref, w2_ref,
               b2_ref, out_ref):
    x = x_ref[...]
    s = (agga_ref[0] + agga_ref[1]) + (aggb_ref[0] + aggb_ref[1])
    h = (jnp.dot(x, w1a_ref[...], preferred_element_type=jnp.float32)
         + jnp.dot(s, w1b_ref[...], preferred_element_type=jnp.float32)
         + b1_ref[...])
    h = jnp.maximum(h, 0.0)
    out_ref[...] = (x + jnp.dot(h, w2_ref[...],
                                preferred_element_type=jnp.float32)
                    + b2_ref[...])


_node_call = pl.pallas_call(
    _node_body,
    out_shape=jax.ShapeDtypeStruct((N_NODES_K, LAT), jnp.float32),
)


def kernel(x, edge_index, edge_attr, eW1, eb1, eW2, eb2, nW1, nb1, nW2, nb2):
    row_a = edge_index[0, :N_EDGES_H]
    row_b = edge_index[0, N_EDGES_H:]
    col_a = edge_index[1, :N_EDGES_H]
    col_b = edge_index[1, N_EDGES_H:]
    e_a = edge_attr[:N_EDGES_H]
    e_b = edge_attr[N_EDGES_H:]
    for l in range(2):
        w1a = eW1[l, :LAT]
        w1b = eW1[l, LAT:2 * LAT]
        w1c = eW1[l, 2 * LAT:]
        eb1l = eb1[l].reshape(1, LAT)
        eb2l = eb2[l].reshape(1, LAT)
        xa, xb = _pre_call(x, w1a, w1b, eb1l)
        g_a = _gather_call(xa, xb, row_a, col_a)
        g_b = _gather_dep_call(xa, xb, row_b, col_b, g_a)
        e_a = _edge_call(g_a, e_a, w1c, eW2[l], eb2l)
        agg_a = _scatter_call(e_a, col_a)
        e_b = _edge_call(g_b, e_b, w1c, eW2[l], eb2l)
        agg_b = _scatter_dep_call(e_b, col_b, agg_a)
        x = _node_call(x, agg_a, agg_b, nW1[l, :LAT], nW1[l, LAT:],
                       nb1[l].reshape(1, LAT), nW2[l],
                       nb2[l].reshape(1, LAT))
    return (x, jnp.concatenate([e_a, e_b], axis=0))


# speedup vs baseline: 1.0489x; 1.0489x over previous
import functools

import jax
import jax.numpy as jnp
from jax import lax
from jax.experimental import pallas as pl
from jax.experimental.pallas import tpu as pltpu
from jax.experimental.pallas import tpu_sc as plsc

LAT = 128
N_NODES_K = 10000
N_EDGES_K = 320000

NC, NS = 2, 16
NW = NC * NS
CHUNK = 128
N_CHUNKS = N_EDGES_K // CHUNK
CPW = (N_CHUNKS + NW - 1) // NW
NFULL = N_NODES_K // CHUNK
NTAIL = N_NODES_K - NFULL * CHUNK
NSLICE = NFULL + 1
SPS = (NSLICE + NS - 1) // NS
ZROWS = 16
NZ = N_NODES_K // ZROWS
ZPS = (NZ + NS - 1) // NS

_SC_MESH = plsc.VectorSubcoreMesh(
    core_axis_name="c", subcore_axis_name="s", num_cores=NC, num_subcores=NS)


def _pre_body(x_ref, wa_ref, wb_ref, b1_ref, xa_ref, xb_ref):
    x = x_ref[...]
    xa_ref[...] = jnp.dot(x, wa_ref[...], preferred_element_type=jnp.float32)
    xb_ref[...] = (jnp.dot(x, wb_ref[...], preferred_element_type=jnp.float32)
                   + b1_ref[...])


_pre_call = pl.pallas_call(
    _pre_body,
    out_shape=(jax.ShapeDtypeStruct((N_NODES_K, LAT), jnp.float32),
               jax.ShapeDtypeStruct((N_NODES_K, LAT), jnp.float32)),
)


LROWS = 640
LTAIL = N_NODES_K - 15 * LROWS


PCH = 80
HW = PCH // 2


def _gather_body(xa_hbm, xb_hbm, row_hbm, col_hbm, g_hbm,
                 xa_sh, ridx, cidx, b0, b1, s0, s1, sw):
    sid = lax.axis_index("s")
    wid = sid * NC + lax.axis_index("c")

    c0 = jnp.minimum(wid * CPW, N_CHUNKS - PCH)

    @pl.when(sid < NS - 1)
    def _():
        pltpu.sync_copy(xa_hbm.at[pl.ds(sid * LROWS, LROWS)],
                        xa_sh.at[pl.ds(sid * LROWS, LROWS)])

    @pl.when(sid == NS - 1)
    def _():
        pltpu.sync_copy(xa_hbm.at[pl.ds((NS - 1) * LROWS, LTAIL)],
                        xa_sh.at[pl.ds((NS - 1) * LROWS, LTAIL)])

    plsc.subcore_barrier()

    bufs = (b0, b1)
    sems = (s0, s1)

    for h in range(2):
        ch = c0 + h * HW
        pltpu.sync_copy(row_hbm.at[pl.ds(ch * CHUNK, HW * CHUNK)], ridx)
        pltpu.sync_copy(col_hbm.at[pl.ds(ch * CHUNK, HW * CHUNK)], cidx)

        def j_body(j, carry):
            adds = []
            for b in range(2):
                i = j * 2 + b
                ri = ridx.at[pl.ds(i * CHUNK, CHUNK)]
                ci = cidx.at[pl.ds(i * CHUNK, CHUNK)]
                pltpu.sync_copy(xa_sh.at[ri], bufs[b])
                adds.append(pltpu.async_copy(xb_hbm.at[ci], bufs[b],
                                             sems[b], add=True))
            adds[0].wait()
            base0 = (ch + j * 2) * CHUNK
            w0 = pltpu.async_copy(b0, g_hbm.at[pl.ds(base0, CHUNK)], sw)
            adds[1].wait()
            pltpu.sync_copy(b1, g_hbm.at[pl.ds(base0 + CHUNK, CHUNK)])
            w0.wait()
            return carry

        lax.fori_loop(0, HW // 2, j_body, 0)


_gather_call = pl.kernel(
    _gather_body,
    out_type=jax.ShapeDtypeStruct((N_EDGES_K, LAT), jnp.float32),
    mesh=_SC_MESH,
    scratch_types=(
        pltpu.VMEM_SHARED((N_NODES_K, LAT), jnp.float32),
        pltpu.VMEM((HW * CHUNK,), jnp.int32),
        pltpu.VMEM((HW * CHUNK,), jnp.int32),
        pltpu.VMEM((CHUNK, LAT), jnp.float32),
        pltpu.VMEM((CHUNK, LAT), jnp.float32),
        pltpu.SemaphoreType.DMA,
        pltpu.SemaphoreType.DMA,
        pltpu.SemaphoreType.DMA,
    ),
)


EBLK = 4000


def _edge_body(g_ref, e_ref, w1c_ref, w2_ref, b2_ref, out_ref):
    e = e_ref[...]
    h = jnp.dot(e.astype(jnp.bfloat16), w1c_ref[...].astype(jnp.bfloat16),
                preferred_element_type=jnp.float32)
    h = jnp.maximum(h + g_ref[...].astype(jnp.float32), 0.0)
    out_ref[...] = (e + jnp.dot(h.astype(jnp.bfloat16),
                                w2_ref[...].astype(jnp.bfloat16),
                                preferred_element_type=jnp.float32)
                    + b2_ref[...])


_edge_call = pl.pallas_call(
    _edge_body,
    grid=(N_EDGES_K // EBLK,),
    in_specs=[
        pl.BlockSpec((EBLK, LAT), lambda i: (i, 0)),
        pl.BlockSpec((EBLK, LAT), lambda i: (i, 0)),
        pl.BlockSpec((LAT, LAT), lambda i: (0, 0)),
        pl.BlockSpec((LAT, LAT), lambda i: (0, 0)),
        pl.BlockSpec((1, LAT), lambda i: (0, 0)),
    ],
    out_specs=pl.BlockSpec((EBLK, LAT), lambda i: (i, 0)),
    out_shape=jax.ShapeDtypeStruct((N_EDGES_K, LAT), jnp.float32),
    compiler_params=pltpu.CompilerParams(
        dimension_semantics=("parallel",)),
)


def _scatter_body(en_hbm, col2_hbm, out_hbm, acc_shared, cidx,
                  d0, d1, zbuf, t0, t1):
    cid = lax.axis_index("c")
    sid = lax.axis_index("s")
    wid = sid * NC + cid

    def zrow(r, carry):
        for j in range(LAT // 16):
            zbuf[r, pl.ds(j * 16, 16)] = jnp.zeros((16,), jnp.float32)
        return carry

    lax.fori_loop(0, ZROWS, zrow, 0)

    def zslice(i, carry):
        j = i * NS + sid

        @pl.when(j < NZ)
        def _():
            pltpu.sync_copy(zbuf, acc_shared.at[pl.ds(j * ZROWS, ZROWS)])

        return carry

    lax.fori_loop(0, ZPS, zslice, 0)

    c0 = jnp.minimum(wid * CPW, N_CHUNKS - CPW)
    off = wid * CPW - c0
    pltpu.sync_copy(col2_hbm.at[pl.ds(c0 * CHUNK, CPW * CHUNK)], cidx)
    plsc.subcore_barrier()

    dbufs = (d0, d1)
    dsems = (t0, t1)

    def j_body(j, carry):
        loads = []
        for b in range(2):
            c = wid * CPW + j * 2 + b
            base = jnp.minimum(c, N_CHUNKS - 1) * CHUNK
            loads.append(pltpu.async_copy(en_hbm.at[pl.ds(base, CHUNK)],
                                          dbufs[b], dsems[b]))
        for b in range(2):
            i = j * 2 + b
            c = wid * CPW + i
            loads[b].wait()

            @pl.when((i < CPW) & (c < N_CHUNKS))
            def _(b=b, i=i):
                pltpu.sync_copy(
                    dbufs[b],
                    acc_shared.at[cidx.at[pl.ds((off + i) * CHUNK, CHUNK)]],
                    add=True)

        return carry

    lax.fori_loop(0, (CPW + 1) // 2, j_body, 0)
    plsc.subcore_barrier()

    def oslice(i, carry):
        j = i * NS + sid

        @pl.when(j < NFULL)
        def _():
            pltpu.sync_copy(acc_shared.at[pl.ds(j * CHUNK, CHUNK)],
                            out_hbm.at[cid, pl.ds(j * CHUNK, CHUNK)])

        @pl.when(j == NFULL)
        def _():
            pltpu.sync_copy(acc_shared.at[pl.ds(NFULL * CHUNK, NTAIL)],
                            out_hbm.at[cid, pl.ds(NFULL * CHUNK, NTAIL)])

        return carry

    lax.fori_loop(0, SPS, oslice, 0)


_scatter_call = pl.kernel(
    _scatter_body,
    out_type=jax.ShapeDtypeStruct((NC, N_NODES_K, LAT), jnp.float32),
    mesh=_SC_MESH,
    scratch_types=(
        pltpu.VMEM_SHARED((N_NODES_K, LAT), jnp.float32),
        pltpu.VMEM((CPW * CHUNK,), jnp.int32),
        pltpu.VMEM((CHUNK, LAT), jnp.float32),
        pltpu.VMEM((CHUNK, LAT), jnp.float32),
        pltpu.VMEM((ZROWS, LAT), jnp.float32),
        pltpu.SemaphoreType.DMA,
        pltpu.SemaphoreType.DMA,
    ),
)


def _node_body(x_ref, agg_ref, w1a_ref, w1b_ref, b1_ref, w2_ref, b2_ref,
               out_ref):
    x = x_ref[...]
    s = agg_ref[0] + agg_ref[1]
    h = (jnp.dot(x, w1a_ref[...], preferred_element_type=jnp.float32)
         + jnp.dot(s, w1b_ref[...], preferred_element_type=jnp.float32)
         + b1_ref[...])
    h = jnp.maximum(h, 0.0)
    out_ref[...] = (x + jnp.dot(h, w2_ref[...],
                                preferred_element_type=jnp.float32)
                    + b2_ref[...])


_node_call = pl.pallas_call(
    _node_body,
    out_shape=jax.ShapeDtypeStruct((N_NODES_K, LAT), jnp.float32),
)


def kernel(x, edge_index, edge_attr, eW1, eb1, eW2, eb2, nW1, nb1, nW2, nb2):
    row2 = edge_index[0]
    col2 = edge_index[1]
    for l in range(2):
        w1a = eW1[l, :LAT]
        w1b = eW1[l, LAT:2 * LAT]
        w1c = eW1[l, 2 * LAT:]
        xa, xb = _pre_call(x, w1a, w1b, eb1[l].reshape(1, LAT))
        g = _gather_call(xa, xb, row2, col2)
        edge_attr = _edge_call(g, edge_attr, w1c, eW2[l],
                               eb2[l].reshape(1, LAT))
        agg2 = _scatter_call(edge_attr, col2)
        x = _node_call(x, agg2, nW1[l, :LAT], nW1[l, LAT:],
                       nb1[l].reshape(1, LAT), nW2[l],
                       nb2[l].reshape(1, LAT))
    return (x, edge_attr)
